# baseline (speedup 1.0000x reference)
import functools

import jax
import jax.numpy as jnp
from jax import lax
from jax.experimental import pallas as pl
from jax.experimental.pallas import tpu as pltpu
from jax.experimental.pallas import tpu_sc as plsc

N_NODES = 10000
N_EDGES = 320000
D_IN = 128

NC = 2
NS = 16
NW = NC * NS
CHUNK = 128
N_PAD = 10112
RPT = N_PAD // NS
EPT_CHUNKS = 84
E_PAD = NW * EPT_CHUNKS * CHUNK


def _sc_scatter(hw, zeros, src_r, dst_r, d, fsplit, mode):
    n_chunks = src_r.shape[1]
    nslots = 1 if mode in ("serial", "gonly", "sonly") else 2
    mesh = plsc.VectorSubcoreMesh(core_axis_name="c", subcore_axis_name="s")

    @functools.partial(
        pl.kernel,
        out_type=jax.ShapeDtypeStruct((NC, N_PAD, d), jnp.float32),
        mesh=mesh,
        scratch_types=[
            pltpu.VMEM((n_chunks, CHUNK), jnp.int32),
            pltpu.VMEM((n_chunks, CHUNK), jnp.int32),
            (pltpu.VMEM((CHUNK, d), jnp.float32) if nslots == 1
             else pltpu.VMEM((nslots, CHUNK, d), jnp.float32)),
            pltpu.VMEM_SHARED((N_PAD, d), jnp.float32),
            pltpu.SemaphoreType.DMA,
            pltpu.SemaphoreType.DMA,
        ],
        compiler_params=pltpu.CompilerParams(use_tc_tiling_on_sc=False),
    )
    def k(hw_hbm, z_hbm, src_hbm, dst_hbm, out_hbm,
          src_v, dst_v, rows_v, acc_sh, gsem0, gsem1):
        c = lax.axis_index("c")
        s = lax.axis_index("s")
        table = hw_hbm.at[c] if fsplit else hw_hbm
        wid = s if fsplit else s * NC + c
        pltpu.sync_copy(src_hbm.at[wid], src_v)
        pltpu.sync_copy(dst_hbm.at[wid], dst_v)
        pltpu.sync_copy(z_hbm.at[pl.ds(s * RPT, RPT)],
                        acc_sh.at[pl.ds(s * RPT, RPT)])
        plsc.subcore_barrier()

        gsems = (gsem0, gsem1)

        def slot(b):
            return rows_v if nslots == 1 else rows_v.at[b]

        def gissue(j, b):
            pltpu.async_copy(table.at[src_v.at[j]], slot(b), gsems[b])

        def gwait(j, b):
            pltpu.make_async_copy(table.at[src_v.at[j]], slot(b),
                                  gsems[b]).wait()

        def ssync(j, b):
            pltpu.sync_copy(slot(b), acc_sh.at[dst_v.at[j]], add=True)

        if mode == "serial":
            @pl.loop(0, n_chunks)
            def _(j):
                pltpu.async_copy(table.at[src_v.at[j]], rows_v,
                                 gsem0).wait()
                ssync(j, 0)
        elif mode == "gonly":
            @pl.loop(0, n_chunks)
            def _(j):
                gissue(j, 0)
                gwait(j, 0)
        elif mode == "sonly":
            @pl.loop(0, n_chunks)
            def _(j):
                ssync(j, 0)
        else:
            gissue(0, 0)

            @pl.loop(0, n_chunks - 2, step=2)
            def _(j):
                gissue(j + 1, 1)
                gwait(j, 0)
                ssync(j, 0)
                gissue(j + 2, 0)
                gwait(j + 1, 1)
                ssync(j + 1, 1)

            jj = n_chunks - 2
            gissue(jj + 1, 1)
            gwait(jj, 0)
            ssync(jj, 0)
            gwait(jj + 1, 1)
            ssync(jj + 1, 1)

        plsc.subcore_barrier()
        pltpu.sync_copy(acc_sh.at[pl.ds(s * RPT, RPT)],
                        out_hbm.at[c].at[pl.ds(s * RPT, RPT)])

    return k(hw, zeros, src_r, dst_r)


def _tc_scale(h, wcol, split):
    def body(h_ref, wc_ref, o_ref):
        hw = wc_ref[...] * h_ref[...]
        if split:
            dh = hw.shape[1] // 2
            o_ref[0] = hw[:, :dh]
            o_ref[1] = hw[:, dh:]
        else:
            o_ref[...] = hw

    n, dim = h.shape
    shp = (2, n, dim // 2) if split else (n, dim)
    return pl.pallas_call(
        body,
        out_shape=jax.ShapeDtypeStruct(shp, jnp.float32),
    )(h, wcol)


def _tc_layer(p, h, w_mat, b, wcol, concat, act):
    def body(p_ref, h_ref, w_ref, b_ref, wc_ref, hn_ref, hwn_ref):
        if concat:
            x = jnp.concatenate([p_ref[0], p_ref[1]], axis=1) + h_ref[...]
        else:
            x = p_ref[0] + p_ref[1] + h_ref[...]
        y = jnp.dot(x, w_ref[...], preferred_element_type=jnp.float32) + b_ref[...]
        y = jnp.maximum(y, 0.0) if act == "relu" else jax.nn.sigmoid(y)
        hn_ref[...] = y
        hwn_ref[...] = wc_ref[...] * y

    d = w_mat.shape[1]
    return pl.pallas_call(
        body,
        out_shape=[
            jax.ShapeDtypeStruct((N_PAD, d), jnp.float32),
            jax.ShapeDtypeStruct((N_PAD, d), jnp.float32),
        ],
    )(p, h, w_mat, b, wcol)


def _tc_last(p, h, w_mat, b):
    def body(p_ref, h_ref, w_ref, b_ref, o_ref):
        x = p_ref[0] + p_ref[1] + h_ref[...]
        o_ref[...] = jax.nn.sigmoid(
            jnp.dot(x, w_ref[...], preferred_element_type=jnp.float32)
            + b_ref[...])

    d = w_mat.shape[1]
    return pl.pallas_call(
        body,
        out_shape=jax.ShapeDtypeStruct((N_PAD, d), jnp.float32),
    )(p, h, w_mat, b)


MODE1 = "serial"
MODE23 = "serial"
FSPLIT1 = False


def kernel(structure, H, input_weight, W1, b1, W2, b2, W3, b3):
    src = structure[0]
    dst = structure[1]
    pad = E_PAD - N_EDGES
    fill = jnp.full((pad,), N_NODES, jnp.int32)
    src_flat = jnp.concatenate([src, fill])
    dst_flat = jnp.concatenate([dst, fill])
    if FSPLIT1:
        src_r1 = src_flat.reshape(NS, 2 * EPT_CHUNKS, CHUNK)
        dst_r1 = dst_flat.reshape(NS, 2 * EPT_CHUNKS, CHUNK)
    else:
        src_r1 = src_flat.reshape(NW, EPT_CHUNKS, CHUNK)
        dst_r1 = dst_flat.reshape(NW, EPT_CHUNKS, CHUNK)
    src_r = src_flat.reshape(NW, EPT_CHUNKS, CHUNK)
    dst_r = dst_flat.reshape(NW, EPT_CHUNKS, CHUNK)

    h_pad = jnp.zeros((N_PAD, D_IN), jnp.float32).at[:N_NODES].set(H)
    wcol = jnp.zeros((N_PAD, 1), jnp.float32).at[:N_NODES, 0].set(input_weight)
    z = jnp.zeros((N_PAD, D_IN), jnp.float32)

    hw1 = _tc_scale(h_pad, wcol, FSPLIT1)
    d1 = 64 if FSPLIT1 else 128
    p1 = _sc_scatter(hw1, z[:, :d1], src_r1, dst_r1, d1, FSPLIT1, MODE1)
    h1, hw2 = _tc_layer(p1, h_pad, W1, b1.reshape(1, -1), wcol, FSPLIT1, "relu")
    p2 = _sc_scatter(hw2, z[:, :32], src_r, dst_r, 32, False, MODE23)
    h2, hw3 = _tc_layer(p2, h1, W2, b2.reshape(1, -1), wcol, False, "relu")
    p3 = _sc_scatter(hw3, z[:, :16], src_r, dst_r, 16, False, MODE23)
    out = _tc_last(p3, h2, W3, b3.reshape(1, -1))
    return out[:N_NODES]


# speedup vs baseline: 3.2337x; 3.2337x over previous
import functools

import jax
import jax.numpy as jnp
from jax import lax
from jax.experimental import pallas as pl
from jax.experimental.pallas import tpu as pltpu
from jax.experimental.pallas import tpu_sc as plsc

N_NODES = 10000
N_EDGES = 320000
D_IN = 128

NC = 2
NS = 16
NW = NC * NS
CHUNK = 128
N_PAD = 10112
RPT = N_PAD // NS
EPT_CHUNKS = 84
E_PAD = NW * EPT_CHUNKS * CHUNK


def _sc_scatter(hw, zeros, src_r, dst_r, d, fsplit, mode):
    n_chunks = src_r.shape[1]
    nslots = 1 if mode in ("serial", "gonly", "sonly") else 2
    mesh = plsc.VectorSubcoreMesh(core_axis_name="c", subcore_axis_name="s")

    @functools.partial(
        pl.kernel,
        out_type=jax.ShapeDtypeStruct((NC, N_PAD, d), jnp.float32),
        mesh=mesh,
        scratch_types=[
            pltpu.VMEM((n_chunks, CHUNK), jnp.int32),
            pltpu.VMEM((n_chunks, CHUNK), jnp.int32),
            (pltpu.VMEM((CHUNK, d), jnp.float32) if nslots == 1
             else pltpu.VMEM((nslots, CHUNK, d), jnp.float32)),
            pltpu.VMEM_SHARED((N_PAD, d), jnp.float32),
            pltpu.SemaphoreType.DMA,
            pltpu.SemaphoreType.DMA,
        ],
        compiler_params=pltpu.CompilerParams(use_tc_tiling_on_sc=False),
    )
    def k(hw_hbm, z_hbm, src_hbm, dst_hbm, out_hbm,
          src_v, dst_v, rows_v, acc_sh, gsem0, gsem1):
        c = lax.axis_index("c")
        s = lax.axis_index("s")
        table = hw_hbm.at[c] if fsplit else hw_hbm
        wid = s if fsplit else s * NC + c
        pltpu.sync_copy(src_hbm.at[wid], src_v)
        pltpu.sync_copy(dst_hbm.at[wid], dst_v)
        pltpu.sync_copy(z_hbm.at[pl.ds(s * RPT, RPT)],
                        acc_sh.at[pl.ds(s * RPT, RPT)])
        plsc.subcore_barrier()

        gsems = (gsem0, gsem1)

        def slot(b):
            return rows_v if nslots == 1 else rows_v.at[b]

        def gissue(j, b):
            pltpu.async_copy(table.at[src_v.at[j]], slot(b), gsems[b])

        def gwait(j, b):
            pltpu.make_async_copy(table.at[src_v.at[j]], slot(b),
                                  gsems[b]).wait()

        def ssync(j, b):
            pltpu.sync_copy(slot(b), acc_sh.at[dst_v.at[j]], add=True)

        if mode == "serial":
            @pl.loop(0, n_chunks)
            def _(j):
                pltpu.async_copy(table.at[src_v.at[j]], rows_v,
                                 gsem0).wait()
                ssync(j, 0)
        elif mode == "gonly":
            @pl.loop(0, n_chunks)
            def _(j):
                gissue(j, 0)
                gwait(j, 0)
        elif mode == "sonly":
            @pl.loop(0, n_chunks)
            def _(j):
                ssync(j, 0)
        else:
            gissue(0, 0)

            @pl.loop(0, n_chunks - 2, step=2)
            def _(j):
                gissue(j + 1, 1)
                gwait(j, 0)
                ssync(j, 0)
                gissue(j + 2, 0)
                gwait(j + 1, 1)
                ssync(j + 1, 1)

            jj = n_chunks - 2
            gissue(jj + 1, 1)
            gwait(jj, 0)
            ssync(jj, 0)
            gwait(jj + 1, 1)
            ssync(jj + 1, 1)

        plsc.subcore_barrier()
        pltpu.sync_copy(acc_sh.at[pl.ds(s * RPT, RPT)],
                        out_hbm.at[c].at[pl.ds(s * RPT, RPT)])

    return k(hw, zeros, src_r, dst_r)


def _tc_scale(h, wcol, split):
    def body(h_ref, wc_ref, o_ref):
        hw = wc_ref[...] * h_ref[...]
        if split:
            dh = hw.shape[1] // 2
            o_ref[0] = hw[:, :dh]
            o_ref[1] = hw[:, dh:]
        else:
            o_ref[...] = hw

    n, dim = h.shape
    shp = (2, n, dim // 2) if split else (n, dim)
    return pl.pallas_call(
        body,
        out_shape=jax.ShapeDtypeStruct(shp, jnp.float32),
    )(h, wcol)


def _tc_layer(p, h, w_mat, b, wcol, concat, act):
    def body(p_ref, h_ref, w_ref, b_ref, wc_ref, hn_ref, hwn_ref):
        if concat:
            x = jnp.concatenate([p_ref[0], p_ref[1]], axis=1) + h_ref[...]
        else:
            x = p_ref[0] + p_ref[1] + h_ref[...]
        y = jnp.dot(x, w_ref[...], preferred_element_type=jnp.float32) + b_ref[...]
        y = jnp.maximum(y, 0.0) if act == "relu" else jax.nn.sigmoid(y)
        hn_ref[...] = y
        hwn_ref[...] = wc_ref[...] * y

    d = w_mat.shape[1]
    return pl.pallas_call(
        body,
        out_shape=[
            jax.ShapeDtypeStruct((N_PAD, d), jnp.float32),
            jax.ShapeDtypeStruct((N_PAD, d), jnp.float32),
        ],
    )(p, h, w_mat, b, wcol)


def _tc_last(p, h, w_mat, b):
    def body(p_ref, h_ref, w_ref, b_ref, o_ref):
        x = p_ref[0] + p_ref[1] + h_ref[...]
        o_ref[...] = jax.nn.sigmoid(
            jnp.dot(x, w_ref[...], preferred_element_type=jnp.float32)
            + b_ref[...])

    d = w_mat.shape[1]
    return pl.pallas_call(
        body,
        out_shape=jax.ShapeDtypeStruct((N_PAD, d), jnp.float32),
    )(p, h, w_mat, b)


MODE1 = "serial"
MODE23 = "serial"
FSPLIT1 = False


def kernel(structure, H, input_weight, W1, b1, W2, b2, W3, b3):
    src = structure[0]
    dst = structure[1]
    pad = E_PAD - N_EDGES
    fill = N_NODES + (jnp.arange(pad, dtype=jnp.int32) % (N_PAD - N_NODES))
    src_flat = jnp.concatenate([src, fill])
    dst_flat = jnp.concatenate([dst, fill])
    if FSPLIT1:
        src_r1 = src_flat.reshape(NS, 2 * EPT_CHUNKS, CHUNK)
        dst_r1 = dst_flat.reshape(NS, 2 * EPT_CHUNKS, CHUNK)
    else:
        src_r1 = src_flat.reshape(NW, EPT_CHUNKS, CHUNK)
        dst_r1 = dst_flat.reshape(NW, EPT_CHUNKS, CHUNK)
    src_r = src_flat.reshape(NW, EPT_CHUNKS, CHUNK)
    dst_r = dst_flat.reshape(NW, EPT_CHUNKS, CHUNK)

    h_pad = jnp.zeros((N_PAD, D_IN), jnp.float32).at[:N_NODES].set(H)
    wcol = jnp.zeros((N_PAD, 1), jnp.float32).at[:N_NODES, 0].set(input_weight)
    z = jnp.zeros((N_PAD, D_IN), jnp.float32)

    hw1 = _tc_scale(h_pad, wcol, FSPLIT1)
    d1 = 64 if FSPLIT1 else 128
    p1 = _sc_scatter(hw1, z[:, :d1], src_r1, dst_r1, d1, FSPLIT1, MODE1)
    h1, hw2 = _tc_layer(p1, h_pad, W1, b1.reshape(1, -1), wcol, FSPLIT1, "relu")
    p2 = _sc_scatter(hw2, z[:, :32], src_r, dst_r, 32, False, MODE23)
    h2, hw3 = _tc_layer(p2, h1, W2, b2.reshape(1, -1), wcol, False, "relu")
    p3 = _sc_scatter(hw3, z[:, :16], src_r, dst_r, 16, False, MODE23)
    out = _tc_last(p3, h2, W3, b3.reshape(1, -1))
    return out[:N_NODES]


# speedup vs baseline: 3.8414x; 1.1879x over previous
import functools

import jax
import jax.numpy as jnp
from jax import lax
from jax.experimental import pallas as pl
from jax.experimental.pallas import tpu as pltpu
from jax.experimental.pallas import tpu_sc as plsc

N_NODES = 10000
N_EDGES = 320000
D_IN = 128

NC = 2
NS = 16
NW = NC * NS
CHUNK = 128
N_PAD = 10112
RPT = N_PAD // NS
EPT_CHUNKS = 84
E_PAD = NW * EPT_CHUNKS * CHUNK


def _sc_scatter(hw, zeros, src_r, dst_r, d, fsplit, mode):
    n_chunks = src_r.shape[1]
    nslots = 1 if mode in ("serial", "gonly", "sonly") else 2
    mesh = plsc.VectorSubcoreMesh(core_axis_name="c", subcore_axis_name="s")

    @functools.partial(
        pl.kernel,
        out_type=jax.ShapeDtypeStruct((NC, N_PAD, d), jnp.float32),
        mesh=mesh,
        scratch_types=[
            pltpu.VMEM((n_chunks, CHUNK), jnp.int32),
            pltpu.VMEM((n_chunks, CHUNK), jnp.int32),
            (pltpu.VMEM((CHUNK, d), jnp.float32) if nslots == 1
             else pltpu.VMEM((nslots, CHUNK, d), jnp.float32)),
            pltpu.VMEM_SHARED((N_PAD, d), jnp.float32),
            pltpu.SemaphoreType.DMA,
            pltpu.SemaphoreType.DMA,
        ],
        compiler_params=pltpu.CompilerParams(use_tc_tiling_on_sc=False),
    )
    def k(hw_hbm, z_hbm, src_hbm, dst_hbm, out_hbm,
          src_v, dst_v, rows_v, acc_sh, gsem0, gsem1):
        c = lax.axis_index("c")
        s = lax.axis_index("s")
        table = hw_hbm.at[c] if fsplit else hw_hbm
        wid = s if fsplit else s * NC + c
        pltpu.sync_copy(src_hbm.at[wid], src_v)
        pltpu.sync_copy(dst_hbm.at[wid], dst_v)
        pltpu.sync_copy(z_hbm.at[pl.ds(s * RPT, RPT)],
                        acc_sh.at[pl.ds(s * RPT, RPT)])
        plsc.subcore_barrier()

        gsems = (gsem0, gsem1)

        def slot(b):
            return rows_v if nslots == 1 else rows_v.at[b]

        def gissue(j, b):
            pltpu.async_copy(table.at[src_v.at[j]], slot(b), gsems[b])

        def gwait(j, b):
            pltpu.make_async_copy(table.at[src_v.at[j]], slot(b),
                                  gsems[b]).wait()

        def ssync(j, b):
            pltpu.sync_copy(slot(b), acc_sh.at[dst_v.at[j]], add=True)

        if mode == "serial":
            @pl.loop(0, n_chunks)
            def _(j):
                pltpu.async_copy(table.at[src_v.at[j]], rows_v,
                                 gsem0).wait()
                ssync(j, 0)
        elif mode == "gonly":
            @pl.loop(0, n_chunks)
            def _(j):
                gissue(j, 0)
                gwait(j, 0)
        elif mode == "sonly":
            @pl.loop(0, n_chunks)
            def _(j):
                ssync(j, 0)
        else:
            gissue(0, 0)

            @pl.loop(0, n_chunks - 2, step=2)
            def _(j):
                gissue(j + 1, 1)
                gwait(j, 0)
                ssync(j, 0)
                gissue(j + 2, 0)
                gwait(j + 1, 1)
                ssync(j + 1, 1)

            jj = n_chunks - 2
            gissue(jj + 1, 1)
            gwait(jj, 0)
            ssync(jj, 0)
            gwait(jj + 1, 1)
            ssync(jj + 1, 1)

        plsc.subcore_barrier()
        pltpu.sync_copy(acc_sh.at[pl.ds(s * RPT, RPT)],
                        out_hbm.at[c].at[pl.ds(s * RPT, RPT)])

    return k(hw, zeros, src_r, dst_r)


def _tc_scale(h, wcol, split):
    def body(h_ref, wc_ref, o_ref):
        hw = wc_ref[...] * h_ref[...]
        if split:
            dh = hw.shape[1] // 2
            o_ref[0] = hw[:, :dh]
            o_ref[1] = hw[:, dh:]
        else:
            o_ref[...] = hw

    n, dim = h.shape
    shp = (2, n, dim // 2) if split else (n, dim)
    return pl.pallas_call(
        body,
        out_shape=jax.ShapeDtypeStruct(shp, jnp.float32),
    )(h, wcol)


def _tc_layer(p, h, w_mat, b, wcol, concat, act):
    def body(p_ref, h_ref, w_ref, b_ref, wc_ref, hn_ref, hwn_ref):
        if concat:
            x = jnp.concatenate([p_ref[0], p_ref[1]], axis=1) + h_ref[...]
        else:
            x = p_ref[0] + p_ref[1] + h_ref[...]
        y = jnp.dot(x, w_ref[...], preferred_element_type=jnp.float32) + b_ref[...]
        y = jnp.maximum(y, 0.0) if act == "relu" else jax.nn.sigmoid(y)
        hn_ref[...] = y
        hwn_ref[...] = wc_ref[...] * y

    d = w_mat.shape[1]
    return pl.pallas_call(
        body,
        out_shape=[
            jax.ShapeDtypeStruct((N_PAD, d), jnp.float32),
            jax.ShapeDtypeStruct((N_PAD, d), jnp.float32),
        ],
    )(p, h, w_mat, b, wcol)


def _tc_last(p, h, w_mat, b):
    def body(p_ref, h_ref, w_ref, b_ref, o_ref):
        x = p_ref[0] + p_ref[1] + h_ref[...]
        o_ref[...] = jax.nn.sigmoid(
            jnp.dot(x, w_ref[...], preferred_element_type=jnp.float32)
            + b_ref[...])

    d = w_mat.shape[1]
    return pl.pallas_call(
        body,
        out_shape=jax.ShapeDtypeStruct((N_PAD, d), jnp.float32),
    )(p, h, w_mat, b)


MODE1 = "serial"
MODE23 = "pipe2"
FSPLIT1 = False


def kernel(structure, H, input_weight, W1, b1, W2, b2, W3, b3):
    src = structure[0]
    dst = structure[1]
    pad = E_PAD - N_EDGES
    fill = N_NODES + (jnp.arange(pad, dtype=jnp.int32) % (N_PAD - N_NODES))
    src_flat = jnp.concatenate([src, fill])
    dst_flat = jnp.concatenate([dst, fill])
    if FSPLIT1:
        src_r1 = src_flat.reshape(NS, 2 * EPT_CHUNKS, CHUNK)
        dst_r1 = dst_flat.reshape(NS, 2 * EPT_CHUNKS, CHUNK)
    else:
        src_r1 = src_flat.reshape(NW, EPT_CHUNKS, CHUNK)
        dst_r1 = dst_flat.reshape(NW, EPT_CHUNKS, CHUNK)
    src_r = src_flat.reshape(NW, EPT_CHUNKS, CHUNK)
    dst_r = dst_flat.reshape(NW, EPT_CHUNKS, CHUNK)

    h_pad = jnp.zeros((N_PAD, D_IN), jnp.float32).at[:N_NODES].set(H)
    wcol = jnp.zeros((N_PAD, 1), jnp.float32).at[:N_NODES, 0].set(input_weight)
    z = jnp.zeros((N_PAD, D_IN), jnp.float32)

    hw1 = _tc_scale(h_pad, wcol, FSPLIT1)
    d1 = 64 if FSPLIT1 else 128
    p1 = _sc_scatter(hw1, z[:, :d1], src_r1, dst_r1, d1, FSPLIT1, MODE1)
    h1, hw2 = _tc_layer(p1, h_pad, W1, b1.reshape(1, -1), wcol, FSPLIT1, "relu")
    p2 = _sc_scatter(hw2, z[:, :32], src_r, dst_r, 32, False, MODE23)
    h2, hw3 = _tc_layer(p2, h1, W2, b2.reshape(1, -1), wcol, False, "relu")
    p3 = _sc_scatter(hw3, z[:, :16], src_r, dst_r, 16, False, MODE23)
    out = _tc_last(p3, h2, W3, b3.reshape(1, -1))
    return out[:N_NODES]


# speedup vs baseline: 4.2358x; 1.1027x over previous
import functools

import jax
import jax.numpy as jnp
from jax import lax
from jax.experimental import pallas as pl
from jax.experimental.pallas import tpu as pltpu
from jax.experimental.pallas import tpu_sc as plsc

N_NODES = 10000
N_EDGES = 320000
D_IN = 128

NC = 2
NS = 16
NW = NC * NS
CHUNK = 128
N_PAD = 10112
RPT = N_PAD // NS
EPT_CHUNKS = 84
E_PAD = NW * EPT_CHUNKS * CHUNK


def _sc_scatter(hw, zeros, src_r, dst_r, d, fsplit, mode):
    n_chunks = src_r.shape[1]
    nslots = 1 if mode in ("serial", "gonly", "sonly") else 2
    mesh = plsc.VectorSubcoreMesh(core_axis_name="c", subcore_axis_name="s")

    @functools.partial(
        pl.kernel,
        out_type=jax.ShapeDtypeStruct((NC, N_PAD, d), jnp.float32),
        mesh=mesh,
        scratch_types=[
            pltpu.VMEM((n_chunks, CHUNK), jnp.int32),
            pltpu.VMEM((n_chunks, CHUNK), jnp.int32),
            (pltpu.VMEM((CHUNK, d), jnp.float32) if nslots == 1
             else pltpu.VMEM((nslots, CHUNK, d), jnp.float32)),
            pltpu.VMEM_SHARED((N_PAD, d), jnp.float32),
            pltpu.SemaphoreType.DMA,
            pltpu.SemaphoreType.DMA,
        ],
        compiler_params=pltpu.CompilerParams(use_tc_tiling_on_sc=False),
    )
    def k(hw_hbm, z_hbm, src_hbm, dst_hbm, out_hbm,
          src_v, dst_v, rows_v, acc_sh, gsem0, gsem1):
        c = lax.axis_index("c")
        s = lax.axis_index("s")
        table = hw_hbm.at[c] if fsplit else hw_hbm
        wid = s if fsplit else s * NC + c
        pltpu.sync_copy(src_hbm.at[wid], src_v)
        pltpu.sync_copy(dst_hbm.at[wid], dst_v)
        pltpu.sync_copy(z_hbm.at[pl.ds(s * RPT, RPT)],
                        acc_sh.at[pl.ds(s * RPT, RPT)])
        plsc.subcore_barrier()

        gsems = (gsem0, gsem1)

        def slot(b):
            return rows_v if nslots == 1 else rows_v.at[b]

        def gissue(j, b):
            pltpu.async_copy(table.at[src_v.at[j]], slot(b), gsems[b])

        def gwait(j, b):
            pltpu.make_async_copy(table.at[src_v.at[j]], slot(b),
                                  gsems[b]).wait()

        def ssync(j, b):
            pltpu.sync_copy(slot(b), acc_sh.at[dst_v.at[j]], add=True)

        if mode == "serial":
            @pl.loop(0, n_chunks)
            def _(j):
                pltpu.async_copy(table.at[src_v.at[j]], rows_v,
                                 gsem0).wait()
                ssync(j, 0)
        elif mode == "gonly":
            @pl.loop(0, n_chunks)
            def _(j):
                gissue(j, 0)
                gwait(j, 0)
        elif mode == "sonly":
            @pl.loop(0, n_chunks)
            def _(j):
                ssync(j, 0)
        else:
            gissue(0, 0)

            @pl.loop(0, n_chunks - 2, step=2)
            def _(j):
                gissue(j + 1, 1)
                gwait(j, 0)
                ssync(j, 0)
                gissue(j + 2, 0)
                gwait(j + 1, 1)
                ssync(j + 1, 1)

            jj = n_chunks - 2
            gissue(jj + 1, 1)
            gwait(jj, 0)
            ssync(jj, 0)
            gwait(jj + 1, 1)
            ssync(jj + 1, 1)

        plsc.subcore_barrier()
        pltpu.sync_copy(acc_sh.at[pl.ds(s * RPT, RPT)],
                        out_hbm.at[c].at[pl.ds(s * RPT, RPT)])

    return k(hw, zeros, src_r, dst_r)


def _tc_scale(h, wcol, split):
    def body(h_ref, wc_ref, o_ref):
        hw = wc_ref[...] * h_ref[...]
        if split:
            dh = hw.shape[1] // 2
            o_ref[0] = hw[:, :dh]
            o_ref[1] = hw[:, dh:]
        else:
            o_ref[...] = hw

    n, dim = h.shape
    shp = (2, n, dim // 2) if split else (n, dim)
    return pl.pallas_call(
        body,
        out_shape=jax.ShapeDtypeStruct(shp, jnp.float32),
    )(h, wcol)


def _tc_layer(p, h, w_mat, b, wcol, concat, act):
    def body(p_ref, h_ref, w_ref, b_ref, wc_ref, hn_ref, hwn_ref):
        if concat:
            x = jnp.concatenate([p_ref[0], p_ref[1]], axis=1) + h_ref[...]
        else:
            x = p_ref[0] + p_ref[1] + h_ref[...]
        y = jnp.dot(x, w_ref[...], preferred_element_type=jnp.float32) + b_ref[...]
        y = jnp.maximum(y, 0.0) if act == "relu" else jax.nn.sigmoid(y)
        hn_ref[...] = y
        hwn_ref[...] = wc_ref[...] * y

    d = w_mat.shape[1]
    return pl.pallas_call(
        body,
        out_shape=[
            jax.ShapeDtypeStruct((N_PAD, d), jnp.float32),
            jax.ShapeDtypeStruct((N_PAD, d), jnp.float32),
        ],
    )(p, h, w_mat, b, wcol)


def _tc_last(p, h, w_mat, b):
    def body(p_ref, h_ref, w_ref, b_ref, o_ref):
        x = p_ref[0] + p_ref[1] + h_ref[...]
        o_ref[...] = jax.nn.sigmoid(
            jnp.dot(x, w_ref[...], preferred_element_type=jnp.float32)
            + b_ref[...])

    d = w_mat.shape[1]
    return pl.pallas_call(
        body,
        out_shape=jax.ShapeDtypeStruct((N_PAD, d), jnp.float32),
    )(p, h, w_mat, b)


MODE1 = "pipe2"
MODE23 = "pipe2"
FSPLIT1 = True


def kernel(structure, H, input_weight, W1, b1, W2, b2, W3, b3):
    src = structure[0]
    dst = structure[1]
    pad = E_PAD - N_EDGES
    fill = N_NODES + (jnp.arange(pad, dtype=jnp.int32) % (N_PAD - N_NODES))
    src_flat = jnp.concatenate([src, fill])
    dst_flat = jnp.concatenate([dst, fill])
    if FSPLIT1:
        src_r1 = src_flat.reshape(NS, 2 * EPT_CHUNKS, CHUNK)
        dst_r1 = dst_flat.reshape(NS, 2 * EPT_CHUNKS, CHUNK)
    else:
        src_r1 = src_flat.reshape(NW, EPT_CHUNKS, CHUNK)
        dst_r1 = dst_flat.reshape(NW, EPT_CHUNKS, CHUNK)
    src_r = src_flat.reshape(NW, EPT_CHUNKS, CHUNK)
    dst_r = dst_flat.reshape(NW, EPT_CHUNKS, CHUNK)

    h_pad = jnp.zeros((N_PAD, D_IN), jnp.float32).at[:N_NODES].set(H)
    wcol = jnp.zeros((N_PAD, 1), jnp.float32).at[:N_NODES, 0].set(input_weight)
    z = jnp.zeros((N_PAD, D_IN), jnp.float32)

    hw1 = _tc_scale(h_pad, wcol, FSPLIT1)
    d1 = 64 if FSPLIT1 else 128
    p1 = _sc_scatter(hw1, z[:, :d1], src_r1, dst_r1, d1, FSPLIT1, MODE1)
    h1, hw2 = _tc_layer(p1, h_pad, W1, b1.reshape(1, -1), wcol, FSPLIT1, "relu")
    p2 = _sc_scatter(hw2, z[:, :32], src_r, dst_r, 32, False, MODE23)
    h2, hw3 = _tc_layer(p2, h1, W2, b2.reshape(1, -1), wcol, False, "relu")
    p3 = _sc_scatter(hw3, z[:, :16], src_r, dst_r, 16, False, MODE23)
    out = _tc_last(p3, h2, W3, b3.reshape(1, -1))
    return out[:N_NODES]


# speedup vs baseline: 4.9545x; 1.1697x over previous
import functools

import jax
import jax.numpy as jnp
from jax import lax
from jax.experimental import pallas as pl
from jax.experimental.pallas import tpu as pltpu
from jax.experimental.pallas import tpu_sc as plsc

N_NODES = 10000
N_EDGES = 320000
D_IN = 128

NC = 2
NS = 16
NW = NC * NS
CHUNK = 128
N_PAD = 10112
RPT = N_PAD // NS
EPT_CHUNKS = 84
E_PAD = NW * EPT_CHUNKS * CHUNK


def _sc_scatter(hw, zeros, src_r, dst_r, d, fsplit, mode):
    n_chunks = src_r.shape[1]
    nslots = 1 if mode in ("serial", "gonly", "sonly") else int(mode[4:])
    assert nslots == 1 or n_chunks % nslots == 0
    mesh = plsc.VectorSubcoreMesh(core_axis_name="c", subcore_axis_name="s")

    @functools.partial(
        pl.kernel,
        out_type=jax.ShapeDtypeStruct((NC, N_PAD, d), jnp.float32),
        mesh=mesh,
        scratch_types=[
            pltpu.VMEM((n_chunks, CHUNK), jnp.int32),
            pltpu.VMEM((n_chunks, CHUNK), jnp.int32),
            (pltpu.VMEM((CHUNK, d), jnp.float32) if nslots == 1
             else pltpu.VMEM((nslots, CHUNK, d), jnp.float32)),
            pltpu.VMEM_SHARED((N_PAD, d), jnp.float32),
        ] + [pltpu.SemaphoreType.DMA] * max(nslots, 1),
        compiler_params=pltpu.CompilerParams(use_tc_tiling_on_sc=False),
    )
    def k(hw_hbm, z_hbm, src_hbm, dst_hbm, out_hbm,
          src_v, dst_v, rows_v, acc_sh, *gsems):
        c = lax.axis_index("c")
        s = lax.axis_index("s")
        table = hw_hbm.at[c] if fsplit else hw_hbm
        wid = s if fsplit else s * NC + c
        pltpu.sync_copy(src_hbm.at[wid], src_v)
        pltpu.sync_copy(dst_hbm.at[wid], dst_v)
        pltpu.sync_copy(z_hbm.at[pl.ds(s * RPT, RPT)],
                        acc_sh.at[pl.ds(s * RPT, RPT)])
        plsc.subcore_barrier()

        def slot(b):
            return rows_v if nslots == 1 else rows_v.at[b]

        def gissue(j, b):
            pltpu.async_copy(table.at[src_v.at[j]], slot(b), gsems[b])

        def gwait(j, b):
            pltpu.make_async_copy(table.at[src_v.at[j]], slot(b),
                                  gsems[b]).wait()

        def ssync(j, b):
            pltpu.sync_copy(slot(b), acc_sh.at[dst_v.at[j]], add=True)

        if mode == "serial":
            @pl.loop(0, n_chunks)
            def _(j):
                pltpu.async_copy(table.at[src_v.at[j]], rows_v,
                                 gsems[0]).wait()
                ssync(j, 0)
        elif mode == "gonly":
            @pl.loop(0, n_chunks)
            def _(j):
                gissue(j, 0)
                gwait(j, 0)
        elif mode == "sonly":
            @pl.loop(0, n_chunks)
            def _(j):
                ssync(j, 0)
        else:
            nb = nslots
            for jj in range(nb - 1):
                gissue(jj, jj)

            @pl.loop(0, n_chunks - nb, step=nb)
            def _(j0):
                for b in range(nb):
                    j = j0 + b
                    gissue(j + nb - 1, (b + nb - 1) % nb)
                    gwait(j, b)
                    ssync(j, b)

            for jj in range(n_chunks - nb, n_chunks):
                if jj + nb - 1 < n_chunks:
                    gissue(jj + nb - 1, (jj + nb - 1) % nb)
                gwait(jj, jj % nb)
                ssync(jj, jj % nb)

        plsc.subcore_barrier()
        pltpu.sync_copy(acc_sh.at[pl.ds(s * RPT, RPT)],
                        out_hbm.at[c].at[pl.ds(s * RPT, RPT)])

    return k(hw, zeros, src_r, dst_r)


def _tc_scale(h, wcol, split):
    def body(h_ref, wc_ref, o_ref):
        hw = wc_ref[...] * h_ref[...]
        if split:
            dh = hw.shape[1] // 2
            o_ref[0] = hw[:, :dh]
            o_ref[1] = hw[:, dh:]
        else:
            o_ref[...] = hw

    n, dim = h.shape
    shp = (2, n, dim // 2) if split else (n, dim)
    return pl.pallas_call(
        body,
        out_shape=jax.ShapeDtypeStruct(shp, jnp.float32),
    )(h, wcol)


def _tc_layer(p, h, w_mat, b, wcol, concat, act):
    def body(p_ref, h_ref, w_ref, b_ref, wc_ref, hn_ref, hwn_ref):
        if concat:
            x = jnp.concatenate([p_ref[0], p_ref[1]], axis=1) + h_ref[...]
        else:
            x = p_ref[0] + p_ref[1] + h_ref[...]
        y = jnp.dot(x, w_ref[...], preferred_element_type=jnp.float32) + b_ref[...]
        y = jnp.maximum(y, 0.0) if act == "relu" else jax.nn.sigmoid(y)
        hn_ref[...] = y
        hwn_ref[...] = wc_ref[...] * y

    d = w_mat.shape[1]
    return pl.pallas_call(
        body,
        out_shape=[
            jax.ShapeDtypeStruct((N_PAD, d), jnp.float32),
            jax.ShapeDtypeStruct((N_PAD, d), jnp.float32),
        ],
    )(p, h, w_mat, b, wcol)


def _tc_last(p, h, w_mat, b):
    def body(p_ref, h_ref, w_ref, b_ref, o_ref):
        x = p_ref[0] + p_ref[1] + h_ref[...]
        o_ref[...] = jax.nn.sigmoid(
            jnp.dot(x, w_ref[...], preferred_element_type=jnp.float32)
            + b_ref[...])

    d = w_mat.shape[1]
    return pl.pallas_call(
        body,
        out_shape=jax.ShapeDtypeStruct((N_PAD, d), jnp.float32),
    )(p, h, w_mat, b)


MODE1 = "pipe3"
MODE23 = "pipe3"
FSPLIT1 = True


def kernel(structure, H, input_weight, W1, b1, W2, b2, W3, b3):
    src = structure[0]
    dst = structure[1]
    pad = E_PAD - N_EDGES
    fill = N_NODES + (jnp.arange(pad, dtype=jnp.int32) % (N_PAD - N_NODES))
    src_flat = jnp.concatenate([src, fill])
    dst_flat = jnp.concatenate([dst, fill])
    if FSPLIT1:
        src_r1 = src_flat.reshape(NS, 2 * EPT_CHUNKS, CHUNK)
        dst_r1 = dst_flat.reshape(NS, 2 * EPT_CHUNKS, CHUNK)
    else:
        src_r1 = src_flat.reshape(NW, EPT_CHUNKS, CHUNK)
        dst_r1 = dst_flat.reshape(NW, EPT_CHUNKS, CHUNK)
    src_r = src_flat.reshape(NW, EPT_CHUNKS, CHUNK)
    dst_r = dst_flat.reshape(NW, EPT_CHUNKS, CHUNK)

    h_pad = jnp.zeros((N_PAD, D_IN), jnp.float32).at[:N_NODES].set(H)
    wcol = jnp.zeros((N_PAD, 1), jnp.float32).at[:N_NODES, 0].set(input_weight)
    z = jnp.zeros((N_PAD, D_IN), jnp.float32)

    hw1 = _tc_scale(h_pad, wcol, FSPLIT1)
    d1 = 64 if FSPLIT1 else 128
    p1 = _sc_scatter(hw1, z[:, :d1], src_r1, dst_r1, d1, FSPLIT1, MODE1)
    h1, hw2 = _tc_layer(p1, h_pad, W1, b1.reshape(1, -1), wcol, FSPLIT1, "relu")
    p2 = _sc_scatter(hw2, z[:, :32], src_r, dst_r, 32, False, MODE23)
    h2, hw3 = _tc_layer(p2, h1, W2, b2.reshape(1, -1), wcol, False, "relu")
    p3 = _sc_scatter(hw3, z[:, :16], src_r, dst_r, 16, False, MODE23)
    out = _tc_last(p3, h2, W3, b3.reshape(1, -1))
    return out[:N_NODES]


# speedup vs baseline: 5.1640x; 1.0423x over previous
import functools

import jax
import jax.numpy as jnp
from jax import lax
from jax.experimental import pallas as pl
from jax.experimental.pallas import tpu as pltpu
from jax.experimental.pallas import tpu_sc as plsc

N_NODES = 10000
N_EDGES = 320000
D_IN = 128

NC = 2
NS = 16
NW = NC * NS
CHUNK = 128
N_PAD = 10112
RPT = N_PAD // NS
EPT_CHUNKS = 84
E_PAD = NW * EPT_CHUNKS * CHUNK


def _sc_scatter(hw, zeros, src_r, dst_r, d, fsplit, mode):
    n_chunks = src_r.shape[1]
    nslots = 1 if mode in ("serial", "gonly", "sonly") else int(mode[4:])
    assert nslots == 1 or n_chunks % nslots == 0
    mesh = plsc.VectorSubcoreMesh(core_axis_name="c", subcore_axis_name="s")

    @functools.partial(
        pl.kernel,
        out_type=jax.ShapeDtypeStruct((NC, N_PAD, d), jnp.float32),
        mesh=mesh,
        scratch_types=[
            pltpu.VMEM((n_chunks, CHUNK), jnp.int32),
            pltpu.VMEM((n_chunks, CHUNK), jnp.int32),
            (pltpu.VMEM((CHUNK, d), jnp.float32) if nslots == 1
             else pltpu.VMEM((nslots, CHUNK, d), jnp.float32)),
            pltpu.VMEM_SHARED((N_PAD, d), jnp.float32),
        ] + [pltpu.SemaphoreType.DMA] * max(nslots, 1),
        compiler_params=pltpu.CompilerParams(use_tc_tiling_on_sc=False),
    )
    def k(hw_hbm, z_hbm, src_hbm, dst_hbm, out_hbm,
          src_v, dst_v, rows_v, acc_sh, *gsems):
        c = lax.axis_index("c")
        s = lax.axis_index("s")
        table = hw_hbm.at[c] if fsplit else hw_hbm
        wid = s if fsplit else s * NC + c
        pltpu.sync_copy(src_hbm.at[wid], src_v)
        pltpu.sync_copy(dst_hbm.at[wid], dst_v)
        pltpu.sync_copy(z_hbm.at[pl.ds(s * RPT, RPT)],
                        acc_sh.at[pl.ds(s * RPT, RPT)])
        plsc.subcore_barrier()

        def slot(b):
            return rows_v if nslots == 1 else rows_v.at[b]

        def gissue(j, b):
            pltpu.async_copy(table.at[src_v.at[j]], slot(b), gsems[b])

        def gwait(j, b):
            pltpu.make_async_copy(table.at[src_v.at[j]], slot(b),
                                  gsems[b]).wait()

        def ssync(j, b):
            pltpu.sync_copy(slot(b), acc_sh.at[dst_v.at[j]], add=True)

        if mode == "serial":
            @pl.loop(0, n_chunks)
            def _(j):
                pltpu.async_copy(table.at[src_v.at[j]], rows_v,
                                 gsems[0]).wait()
                ssync(j, 0)
        elif mode == "gonly":
            @pl.loop(0, n_chunks)
            def _(j):
                gissue(j, 0)
                gwait(j, 0)
        elif mode == "sonly":
            @pl.loop(0, n_chunks)
            def _(j):
                ssync(j, 0)
        else:
            nb = nslots
            for jj in range(nb - 1):
                gissue(jj, jj)

            @pl.loop(0, n_chunks - nb, step=nb)
            def _(j0):
                for b in range(nb):
                    j = j0 + b
                    gissue(j + nb - 1, (b + nb - 1) % nb)
                    gwait(j, b)
                    ssync(j, b)

            for jj in range(n_chunks - nb, n_chunks):
                if jj + nb - 1 < n_chunks:
                    gissue(jj + nb - 1, (jj + nb - 1) % nb)
                gwait(jj, jj % nb)
                ssync(jj, jj % nb)

        plsc.subcore_barrier()
        pltpu.sync_copy(acc_sh.at[pl.ds(s * RPT, RPT)],
                        out_hbm.at[c].at[pl.ds(s * RPT, RPT)])

    return k(hw, zeros, src_r, dst_r)


def _tc_scale(h, wcol, split):
    def body(h_ref, wc_ref, o_ref):
        hw = wc_ref[...] * h_ref[...]
        if split:
            dh = hw.shape[1] // 2
            o_ref[0] = hw[:, :dh]
            o_ref[1] = hw[:, dh:]
        else:
            o_ref[...] = hw

    n, dim = h.shape
    shp = (2, n, dim // 2) if split else (n, dim)
    return pl.pallas_call(
        body,
        out_shape=jax.ShapeDtypeStruct(shp, jnp.float32),
    )(h, wcol)


def _tc_layer(p, h, w_mat, b, wcol, concat, act):
    def body(p_ref, h_ref, w_ref, b_ref, wc_ref, hn_ref, hwn_ref):
        if concat:
            x = jnp.concatenate([p_ref[0], p_ref[1]], axis=1) + h_ref[...]
        else:
            x = p_ref[0] + p_ref[1] + h_ref[...]
        y = jnp.dot(x, w_ref[...], preferred_element_type=jnp.float32) + b_ref[...]
        y = jnp.maximum(y, 0.0) if act == "relu" else jax.nn.sigmoid(y)
        hn_ref[...] = y
        hwn_ref[...] = wc_ref[...] * y

    d = w_mat.shape[1]
    return pl.pallas_call(
        body,
        out_shape=[
            jax.ShapeDtypeStruct((N_PAD, d), jnp.float32),
            jax.ShapeDtypeStruct((N_PAD, d), jnp.float32),
        ],
    )(p, h, w_mat, b, wcol)


def _tc_last(p, h, w_mat, b):
    def body(p_ref, h_ref, w_ref, b_ref, o_ref):
        x = p_ref[0] + p_ref[1] + h_ref[...]
        o_ref[...] = jax.nn.sigmoid(
            jnp.dot(x, w_ref[...], preferred_element_type=jnp.float32)
            + b_ref[...])

    d = w_mat.shape[1]
    return pl.pallas_call(
        body,
        out_shape=jax.ShapeDtypeStruct((N_PAD, d), jnp.float32),
    )(p, h, w_mat, b)


MODE1 = "pipe4"
MODE23 = "pipe4"
FSPLIT1 = True


def kernel(structure, H, input_weight, W1, b1, W2, b2, W3, b3):
    src = structure[0]
    dst = structure[1]
    pad = E_PAD - N_EDGES
    fill = N_NODES + (jnp.arange(pad, dtype=jnp.int32) % (N_PAD - N_NODES))
    src_flat = jnp.concatenate([src, fill])
    dst_flat = jnp.concatenate([dst, fill])
    if FSPLIT1:
        src_r1 = src_flat.reshape(NS, 2 * EPT_CHUNKS, CHUNK)
        dst_r1 = dst_flat.reshape(NS, 2 * EPT_CHUNKS, CHUNK)
    else:
        src_r1 = src_flat.reshape(NW, EPT_CHUNKS, CHUNK)
        dst_r1 = dst_flat.reshape(NW, EPT_CHUNKS, CHUNK)
    src_r = src_flat.reshape(NW, EPT_CHUNKS, CHUNK)
    dst_r = dst_flat.reshape(NW, EPT_CHUNKS, CHUNK)

    h_pad = jnp.zeros((N_PAD, D_IN), jnp.float32).at[:N_NODES].set(H)
    wcol = jnp.zeros((N_PAD, 1), jnp.float32).at[:N_NODES, 0].set(input_weight)
    z = jnp.zeros((N_PAD, D_IN), jnp.float32)

    hw1 = _tc_scale(h_pad, wcol, FSPLIT1)
    d1 = 64 if FSPLIT1 else 128
    p1 = _sc_scatter(hw1, z[:, :d1], src_r1, dst_r1, d1, FSPLIT1, MODE1)
    h1, hw2 = _tc_layer(p1, h_pad, W1, b1.reshape(1, -1), wcol, FSPLIT1, "relu")
    p2 = _sc_scatter(hw2, z[:, :32], src_r, dst_r, 32, False, MODE23)
    h2, hw3 = _tc_layer(p2, h1, W2, b2.reshape(1, -1), wcol, False, "relu")
    p3 = _sc_scatter(hw3, z[:, :16], src_r, dst_r, 16, False, MODE23)
    out = _tc_last(p3, h2, W3, b3.reshape(1, -1))
    return out[:N_NODES]


# speedup vs baseline: 5.2304x; 1.0129x over previous
import functools

import jax
import jax.numpy as jnp
from jax import lax
from jax.experimental import pallas as pl
from jax.experimental.pallas import tpu as pltpu
from jax.experimental.pallas import tpu_sc as plsc

N_NODES = 10000
N_EDGES = 320000
D_IN = 128

NC = 2
NS = 16
NW = NC * NS
CHUNK = 128
N_PAD = 10112
RPT = N_PAD // NS
EPT_CHUNKS = 84
E_PAD = NW * EPT_CHUNKS * CHUNK


def _sc_scatter(hw, zeros, src_r, dst_r, d, fsplit, mode):
    n_chunks = src_r.shape[1]
    nslots = 1 if mode in ("serial", "gonly", "sonly") else int(mode[4:])
    assert nslots == 1 or n_chunks % nslots == 0
    mesh = plsc.VectorSubcoreMesh(core_axis_name="c", subcore_axis_name="s")

    @functools.partial(
        pl.kernel,
        out_type=jax.ShapeDtypeStruct((NC, N_PAD, d), jnp.float32),
        mesh=mesh,
        scratch_types=[
            pltpu.VMEM((n_chunks, CHUNK), jnp.int32),
            pltpu.VMEM((n_chunks, CHUNK), jnp.int32),
            (pltpu.VMEM((CHUNK, d), jnp.float32) if nslots == 1
             else pltpu.VMEM((nslots, CHUNK, d), jnp.float32)),
            pltpu.VMEM_SHARED((N_PAD, d), jnp.float32),
        ] + [pltpu.SemaphoreType.DMA] * max(nslots, 1),
        compiler_params=pltpu.CompilerParams(use_tc_tiling_on_sc=False),
    )
    def k(hw_hbm, z_hbm, src_hbm, dst_hbm, out_hbm,
          src_v, dst_v, rows_v, acc_sh, *gsems):
        c = lax.axis_index("c")
        s = lax.axis_index("s")
        table = hw_hbm.at[c] if fsplit else hw_hbm
        wid = s if fsplit else s * NC + c
        pltpu.sync_copy(src_hbm.at[wid], src_v)
        pltpu.sync_copy(dst_hbm.at[wid], dst_v)
        pltpu.sync_copy(z_hbm.at[pl.ds(s * RPT, RPT)],
                        acc_sh.at[pl.ds(s * RPT, RPT)])
        plsc.subcore_barrier()

        def slot(b):
            return rows_v if nslots == 1 else rows_v.at[b]

        def gissue(j, b):
            pltpu.async_copy(table.at[src_v.at[j]], slot(b), gsems[b])

        def gwait(j, b):
            pltpu.make_async_copy(table.at[src_v.at[j]], slot(b),
                                  gsems[b]).wait()

        def ssync(j, b):
            pltpu.sync_copy(slot(b), acc_sh.at[dst_v.at[j]], add=True)

        if mode == "serial":
            @pl.loop(0, n_chunks)
            def _(j):
                pltpu.async_copy(table.at[src_v.at[j]], rows_v,
                                 gsems[0]).wait()
                ssync(j, 0)
        elif mode == "gonly":
            @pl.loop(0, n_chunks)
            def _(j):
                gissue(j, 0)
                gwait(j, 0)
        elif mode == "sonly":
            @pl.loop(0, n_chunks)
            def _(j):
                ssync(j, 0)
        else:
            nb = nslots
            for jj in range(nb - 1):
                gissue(jj, jj)

            @pl.loop(0, n_chunks - nb, step=nb)
            def _(j0):
                for b in range(nb):
                    j = j0 + b
                    gissue(j + nb - 1, (b + nb - 1) % nb)
                    gwait(j, b)
                    ssync(j, b)

            for jj in range(n_chunks - nb, n_chunks):
                if jj + nb - 1 < n_chunks:
                    gissue(jj + nb - 1, (jj + nb - 1) % nb)
                gwait(jj, jj % nb)
                ssync(jj, jj % nb)

        plsc.subcore_barrier()
        pltpu.sync_copy(acc_sh.at[pl.ds(s * RPT, RPT)],
                        out_hbm.at[c].at[pl.ds(s * RPT, RPT)])

    return k(hw, zeros, src_r, dst_r)


def _tc_scale(h, wcol, split):
    def body(h_ref, wc_ref, o_ref):
        hw = wc_ref[...] * h_ref[...]
        if split:
            dh = hw.shape[1] // 2
            o_ref[0] = hw[:, :dh]
            o_ref[1] = hw[:, dh:]
        else:
            o_ref[...] = hw

    n, dim = h.shape
    shp = (2, n, dim // 2) if split else (n, dim)
    return pl.pallas_call(
        body,
        out_shape=jax.ShapeDtypeStruct(shp, jnp.float32),
    )(h, wcol)


def _tc_layer(p, h, w_mat, b, wcol, concat, act):
    def body(p_ref, h_ref, w_ref, b_ref, wc_ref, hn_ref, hwn_ref):
        if concat:
            x = jnp.concatenate([p_ref[0], p_ref[1]], axis=1) + h_ref[...]
        else:
            x = p_ref[0] + p_ref[1] + h_ref[...]
        y = jnp.dot(x, w_ref[...], preferred_element_type=jnp.float32) + b_ref[...]
        y = jnp.maximum(y, 0.0) if act == "relu" else jax.nn.sigmoid(y)
        hn_ref[...] = y
        hwn_ref[...] = wc_ref[...] * y

    d = w_mat.shape[1]
    return pl.pallas_call(
        body,
        out_shape=[
            jax.ShapeDtypeStruct((N_PAD, d), jnp.float32),
            jax.ShapeDtypeStruct((N_PAD, d), jnp.float32),
        ],
    )(p, h, w_mat, b, wcol)


def _tc_last(p, h, w_mat, b):
    def body(p_ref, h_ref, w_ref, b_ref, o_ref):
        x = p_ref[0] + p_ref[1] + h_ref[...]
        o_ref[...] = jax.nn.sigmoid(
            jnp.dot(x, w_ref[...], preferred_element_type=jnp.float32)
            + b_ref[...])

    d = w_mat.shape[1]
    return pl.pallas_call(
        body,
        out_shape=jax.ShapeDtypeStruct((N_PAD, d), jnp.float32),
    )(p, h, w_mat, b)


MODE1 = "pipe4"
MODE23 = "pipe6"
FSPLIT1 = True


def kernel(structure, H, input_weight, W1, b1, W2, b2, W3, b3):
    src = structure[0]
    dst = structure[1]
    pad = E_PAD - N_EDGES
    fill = N_NODES + (jnp.arange(pad, dtype=jnp.int32) % (N_PAD - N_NODES))
    src_flat = jnp.concatenate([src, fill])
    dst_flat = jnp.concatenate([dst, fill])
    if FSPLIT1:
        src_r1 = src_flat.reshape(NS, 2 * EPT_CHUNKS, CHUNK)
        dst_r1 = dst_flat.reshape(NS, 2 * EPT_CHUNKS, CHUNK)
    else:
        src_r1 = src_flat.reshape(NW, EPT_CHUNKS, CHUNK)
        dst_r1 = dst_flat.reshape(NW, EPT_CHUNKS, CHUNK)
    src_r = src_flat.reshape(NW, EPT_CHUNKS, CHUNK)
    dst_r = dst_flat.reshape(NW, EPT_CHUNKS, CHUNK)

    h_pad = jnp.zeros((N_PAD, D_IN), jnp.float32).at[:N_NODES].set(H)
    wcol = jnp.zeros((N_PAD, 1), jnp.float32).at[:N_NODES, 0].set(input_weight)
    z = jnp.zeros((N_PAD, D_IN), jnp.float32)

    hw1 = _tc_scale(h_pad, wcol, FSPLIT1)
    d1 = 64 if FSPLIT1 else 128
    p1 = _sc_scatter(hw1, z[:, :d1], src_r1, dst_r1, d1, FSPLIT1, MODE1)
    h1, hw2 = _tc_layer(p1, h_pad, W1, b1.reshape(1, -1), wcol, FSPLIT1, "relu")
    p2 = _sc_scatter(hw2, z[:, :32], src_r, dst_r, 32, False, MODE23)
    h2, hw3 = _tc_layer(p2, h1, W2, b2.reshape(1, -1), wcol, False, "relu")
    p3 = _sc_scatter(hw3, z[:, :16], src_r, dst_r, 16, False, MODE23)
    out = _tc_last(p3, h2, W3, b3.reshape(1, -1))
    return out[:N_NODES]


# speedup vs baseline: 5.3164x; 1.0165x over previous
import functools

import jax
import jax.numpy as jnp
from jax import lax
from jax.experimental import pallas as pl
from jax.experimental.pallas import tpu as pltpu
from jax.experimental.pallas import tpu_sc as plsc

N_NODES = 10000
N_EDGES = 320000
D_IN = 128

NC = 2
NS = 16
NW = NC * NS
CHUNK = 128
N_PAD = 10112
RPT = N_PAD // NS
EPT_CHUNKS = 84
E_PAD = NW * EPT_CHUNKS * CHUNK


def _sc_scatter(hw, zeros, src_r, dst_r, d, fsplit, mode):
    n_chunks = src_r.shape[1]
    nslots = 1 if mode in ("serial", "gonly", "sonly") else int(mode[4:])
    assert nslots == 1 or n_chunks % nslots == 0
    mesh = plsc.VectorSubcoreMesh(core_axis_name="c", subcore_axis_name="s")

    @functools.partial(
        pl.kernel,
        out_type=jax.ShapeDtypeStruct((NC, N_PAD, d), jnp.float32),
        mesh=mesh,
        scratch_types=[
            pltpu.VMEM((n_chunks, CHUNK), jnp.int32),
            pltpu.VMEM((n_chunks, CHUNK), jnp.int32),
            (pltpu.VMEM((CHUNK, d), jnp.float32) if nslots == 1
             else pltpu.VMEM((nslots, CHUNK, d), jnp.float32)),
            pltpu.VMEM_SHARED((N_PAD, d), jnp.float32),
        ] + [pltpu.SemaphoreType.DMA] * max(nslots, 1),
        compiler_params=pltpu.CompilerParams(use_tc_tiling_on_sc=False),
    )
    def k(hw_hbm, z_hbm, src_hbm, dst_hbm, out_hbm,
          src_v, dst_v, rows_v, acc_sh, *gsems):
        c = lax.axis_index("c")
        s = lax.axis_index("s")
        table = hw_hbm.at[c] if fsplit else hw_hbm
        wid = s if fsplit else s * NC + c
        pltpu.sync_copy(src_hbm.at[wid], src_v)
        pltpu.sync_copy(dst_hbm.at[wid], dst_v)
        pltpu.sync_copy(z_hbm.at[pl.ds(s * RPT, RPT)],
                        acc_sh.at[pl.ds(s * RPT, RPT)])
        plsc.subcore_barrier()

        def slot(b):
            return rows_v if nslots == 1 else rows_v.at[b]

        def gissue(j, b):
            pltpu.async_copy(table.at[src_v.at[j]], slot(b), gsems[b])

        def gwait(j, b):
            pltpu.make_async_copy(table.at[src_v.at[j]], slot(b),
                                  gsems[b]).wait()

        def ssync(j, b):
            pltpu.sync_copy(slot(b), acc_sh.at[dst_v.at[j]], add=True)

        if mode == "serial":
            @pl.loop(0, n_chunks)
            def _(j):
                pltpu.async_copy(table.at[src_v.at[j]], rows_v,
                                 gsems[0]).wait()
                ssync(j, 0)
        elif mode == "gonly":
            @pl.loop(0, n_chunks)
            def _(j):
                gissue(j, 0)
                gwait(j, 0)
        elif mode == "sonly":
            @pl.loop(0, n_chunks)
            def _(j):
                ssync(j, 0)
        else:
            nb = nslots
            for jj in range(nb - 1):
                gissue(jj, jj)

            @pl.loop(0, n_chunks - nb, step=nb)
            def _(j0):
                for b in range(nb):
                    j = j0 + b
                    gissue(j + nb - 1, (b + nb - 1) % nb)
                    gwait(j, b)
                    ssync(j, b)

            for jj in range(n_chunks - nb, n_chunks):
                if jj + nb - 1 < n_chunks:
                    gissue(jj + nb - 1, (jj + nb - 1) % nb)
                gwait(jj, jj % nb)
                ssync(jj, jj % nb)

        plsc.subcore_barrier()
        pltpu.sync_copy(acc_sh.at[pl.ds(s * RPT, RPT)],
                        out_hbm.at[c].at[pl.ds(s * RPT, RPT)])

    return k(hw, zeros, src_r, dst_r)


def _tc_scale(h, wcol, split):
    def body(h_ref, wc_ref, o_ref):
        hw = wc_ref[...] * h_ref[...]
        if split:
            dh = hw.shape[1] // 2
            o_ref[0, pl.ds(0, N_NODES), :] = hw[:, :dh]
            o_ref[1, pl.ds(0, N_NODES), :] = hw[:, dh:]
        else:
            o_ref[pl.ds(0, N_NODES), :] = hw

    dim = h.shape[1]
    shp = (2, N_PAD, dim // 2) if split else (N_PAD, dim)
    return pl.pallas_call(
        body,
        out_shape=jax.ShapeDtypeStruct(shp, jnp.float32),
    )(h, wcol)


def _tc_layer(p, h, w_mat, b, wcol, concat, act):
    def body(p_ref, h_ref, w_ref, b_ref, wc_ref, hn_ref, hwn_ref):
        hr = h_ref[pl.ds(0, N_NODES), :]
        if concat:
            x = jnp.concatenate(
                [p_ref[0, pl.ds(0, N_NODES), :],
                 p_ref[1, pl.ds(0, N_NODES), :]], axis=1) + hr
        else:
            x = (p_ref[0, pl.ds(0, N_NODES), :]
                 + p_ref[1, pl.ds(0, N_NODES), :] + hr)
        y = jnp.dot(x, w_ref[...], preferred_element_type=jnp.float32) + b_ref[...]
        y = jnp.maximum(y, 0.0) if act == "relu" else jax.nn.sigmoid(y)
        hn_ref[pl.ds(0, N_NODES), :] = y
        hwn_ref[pl.ds(0, N_NODES), :] = wc_ref[...] * y

    d = w_mat.shape[1]
    return pl.pallas_call(
        body,
        out_shape=[
            jax.ShapeDtypeStruct((N_PAD, d), jnp.float32),
            jax.ShapeDtypeStruct((N_PAD, d), jnp.float32),
        ],
    )(p, h, w_mat, b, wcol)


def _tc_last(p, h, w_mat, b):
    def body(p_ref, h_ref, w_ref, b_ref, o_ref):
        x = (p_ref[0, pl.ds(0, N_NODES), :]
             + p_ref[1, pl.ds(0, N_NODES), :]
             + h_ref[pl.ds(0, N_NODES), :])
        o_ref[...] = jax.nn.sigmoid(
            jnp.dot(x, w_ref[...], preferred_element_type=jnp.float32)
            + b_ref[...])

    d = w_mat.shape[1]
    return pl.pallas_call(
        body,
        out_shape=jax.ShapeDtypeStruct((N_NODES, d), jnp.float32),
    )(p, h, w_mat, b)


MODE1 = "pipe4"
MODE23 = "pipe6"
FSPLIT1 = True


def kernel(structure, H, input_weight, W1, b1, W2, b2, W3, b3):
    src = structure[0]
    dst = structure[1]
    pad = E_PAD - N_EDGES
    fill = N_NODES + (jnp.arange(pad, dtype=jnp.int32) % (N_PAD - N_NODES))
    src_flat = jnp.concatenate([src, fill])
    dst_flat = jnp.concatenate([dst, fill])
    if FSPLIT1:
        src_r1 = src_flat.reshape(NS, 2 * EPT_CHUNKS, CHUNK)
        dst_r1 = dst_flat.reshape(NS, 2 * EPT_CHUNKS, CHUNK)
    else:
        src_r1 = src_flat.reshape(NW, EPT_CHUNKS, CHUNK)
        dst_r1 = dst_flat.reshape(NW, EPT_CHUNKS, CHUNK)
    src_r = src_flat.reshape(NW, EPT_CHUNKS, CHUNK)
    dst_r = dst_flat.reshape(NW, EPT_CHUNKS, CHUNK)

    wcol = input_weight.reshape(N_NODES, 1)
    z = jnp.zeros((N_PAD, D_IN), jnp.float32)

    hw1 = _tc_scale(H, wcol, FSPLIT1)
    d1 = 64 if FSPLIT1 else 128
    p1 = _sc_scatter(hw1, z[:, :d1], src_r1, dst_r1, d1, FSPLIT1, MODE1)
    h1, hw2 = _tc_layer(p1, H, W1, b1.reshape(1, -1), wcol, FSPLIT1, "relu")
    p2 = _sc_scatter(hw2, z[:, :32], src_r, dst_r, 32, False, MODE23)
    h2, hw3 = _tc_layer(p2, h1, W2, b2.reshape(1, -1), wcol, False, "relu")
    p3 = _sc_scatter(hw3, z[:, :16], src_r, dst_r, 16, False, MODE23)
    return _tc_last(p3, h2, W3, b3.reshape(1, -1))


# speedup vs baseline: 5.4322x; 1.0218x over previous
import functools

import jax
import jax.numpy as jnp
from jax import lax
from jax.experimental import pallas as pl
from jax.experimental.pallas import tpu as pltpu
from jax.experimental.pallas import tpu_sc as plsc

N_NODES = 10000
N_EDGES = 320000
D_IN = 128

NC = 2
NS = 16
NW = NC * NS
CHUNK = 128
N_PAD = 10112
RPT = N_PAD // NS
EPT_CHUNKS = 84
E_PAD = NW * EPT_CHUNKS * CHUNK


def _sc_scatter(hw, zeros, src_r, dst_r, d, fsplit, mode):
    n_chunks = src_r.shape[1]
    nslots = 1 if mode in ("serial", "gonly", "sonly") else int(mode[4:])
    assert nslots == 1 or n_chunks % nslots == 0
    mesh = plsc.VectorSubcoreMesh(core_axis_name="c", subcore_axis_name="s")

    @functools.partial(
        pl.kernel,
        out_type=jax.ShapeDtypeStruct((NC, N_PAD, d), jnp.float32),
        mesh=mesh,
        scratch_types=[
            pltpu.VMEM((n_chunks, CHUNK), jnp.int32),
            pltpu.VMEM((n_chunks, CHUNK), jnp.int32),
            (pltpu.VMEM((CHUNK, d), jnp.float32) if nslots == 1
             else pltpu.VMEM((nslots, CHUNK, d), jnp.float32)),
            pltpu.VMEM_SHARED((N_PAD, d), jnp.float32),
        ] + [pltpu.SemaphoreType.DMA] * max(nslots, 1),
        compiler_params=pltpu.CompilerParams(use_tc_tiling_on_sc=False),
    )
    def k(hw_hbm, z_hbm, src_hbm, dst_hbm, out_hbm,
          src_v, dst_v, rows_v, acc_sh, *gsems):
        c = lax.axis_index("c")
        s = lax.axis_index("s")
        table = hw_hbm.at[c] if fsplit else hw_hbm
        wid = s if fsplit else s * NC + c
        cp_src = pltpu.async_copy(src_hbm.at[wid], src_v, gsems[0])
        cp_dst = pltpu.async_copy(dst_hbm.at[wid], dst_v, gsems[1 % len(gsems)])
        pltpu.sync_copy(z_hbm.at[pl.ds(s * RPT, RPT)],
                        acc_sh.at[pl.ds(s * RPT, RPT)])
        cp_src.wait()
        cp_dst.wait()
        plsc.subcore_barrier()

        def slot(b):
            return rows_v if nslots == 1 else rows_v.at[b]

        def gissue(j, b):
            pltpu.async_copy(table.at[src_v.at[j]], slot(b), gsems[b])

        def gwait(j, b):
            pltpu.make_async_copy(table.at[src_v.at[j]], slot(b),
                                  gsems[b]).wait()

        def ssync(j, b):
            pltpu.sync_copy(slot(b), acc_sh.at[dst_v.at[j]], add=True)

        if mode == "serial":
            @pl.loop(0, n_chunks)
            def _(j):
                pltpu.async_copy(table.at[src_v.at[j]], rows_v,
                                 gsems[0]).wait()
                ssync(j, 0)
        elif mode == "gonly":
            @pl.loop(0, n_chunks)
            def _(j):
                gissue(j, 0)
                gwait(j, 0)
        elif mode == "sonly":
            @pl.loop(0, n_chunks)
            def _(j):
                ssync(j, 0)
        else:
            nb = nslots
            for jj in range(nb - 1):
                gissue(jj, jj)

            @pl.loop(0, n_chunks - nb, step=nb)
            def _(j0):
                for b in range(nb):
                    j = j0 + b
                    gissue(j + nb - 1, (b + nb - 1) % nb)
                    gwait(j, b)
                    ssync(j, b)

            for jj in range(n_chunks - nb, n_chunks):
                if jj + nb - 1 < n_chunks:
                    gissue(jj + nb - 1, (jj + nb - 1) % nb)
                gwait(jj, jj % nb)
                ssync(jj, jj % nb)

        plsc.subcore_barrier()
        pltpu.sync_copy(acc_sh.at[pl.ds(s * RPT, RPT)],
                        out_hbm.at[c].at[pl.ds(s * RPT, RPT)])

    return k(hw, zeros, src_r, dst_r)


def _tc_scale(h, wcol, split):
    def body(h_ref, wc_ref, o_ref):
        hw = wc_ref[...] * h_ref[...]
        if split:
            dh = hw.shape[1] // 2
            o_ref[0, pl.ds(0, N_NODES), :] = hw[:, :dh]
            o_ref[1, pl.ds(0, N_NODES), :] = hw[:, dh:]
        else:
            o_ref[pl.ds(0, N_NODES), :] = hw

    dim = h.shape[1]
    shp = (2, N_PAD, dim // 2) if split else (N_PAD, dim)
    return pl.pallas_call(
        body,
        out_shape=jax.ShapeDtypeStruct(shp, jnp.float32),
    )(h, wcol)


def _tc_layer(p, h, w_mat, b, wcol, concat, act):
    def body(p_ref, h_ref, w_ref, b_ref, wc_ref, hn_ref, hwn_ref):
        hr = h_ref[pl.ds(0, N_NODES), :]
        if concat:
            x = jnp.concatenate(
                [p_ref[0, pl.ds(0, N_NODES), :],
                 p_ref[1, pl.ds(0, N_NODES), :]], axis=1) + hr
        else:
            x = (p_ref[0, pl.ds(0, N_NODES), :]
                 + p_ref[1, pl.ds(0, N_NODES), :] + hr)
        y = jnp.dot(x, w_ref[...], preferred_element_type=jnp.float32) + b_ref[...]
        y = jnp.maximum(y, 0.0) if act == "relu" else jax.nn.sigmoid(y)
        hn_ref[pl.ds(0, N_NODES), :] = y
        hwn_ref[pl.ds(0, N_NODES), :] = wc_ref[...] * y

    d = w_mat.shape[1]
    return pl.pallas_call(
        body,
        out_shape=[
            jax.ShapeDtypeStruct((N_PAD, d), jnp.float32),
            jax.ShapeDtypeStruct((N_PAD, d), jnp.float32),
        ],
    )(p, h, w_mat, b, wcol)


def _tc_last(p, h, w_mat, b):
    def body(p_ref, h_ref, w_ref, b_ref, o_ref):
        x = (p_ref[0, pl.ds(0, N_NODES), :]
             + p_ref[1, pl.ds(0, N_NODES), :]
             + h_ref[pl.ds(0, N_NODES), :])
        o_ref[...] = jax.nn.sigmoid(
            jnp.dot(x, w_ref[...], preferred_element_type=jnp.float32)
            + b_ref[...])

    d = w_mat.shape[1]
    return pl.pallas_call(
        body,
        out_shape=jax.ShapeDtypeStruct((N_NODES, d), jnp.float32),
    )(p, h, w_mat, b)


MODE1 = "pipe4"
MODE23 = "pipe6"
FSPLIT1 = True


def kernel(structure, H, input_weight, W1, b1, W2, b2, W3, b3):
    src = structure[0]
    dst = structure[1]
    pad = E_PAD - N_EDGES
    fill = N_NODES + (jnp.arange(pad, dtype=jnp.int32) % (N_PAD - N_NODES))
    src_flat = jnp.concatenate([src, fill])
    dst_flat = jnp.concatenate([dst, fill])
    if FSPLIT1:
        src_r1 = src_flat.reshape(NS, 2 * EPT_CHUNKS, CHUNK)
        dst_r1 = dst_flat.reshape(NS, 2 * EPT_CHUNKS, CHUNK)
    else:
        src_r1 = src_flat.reshape(NW, EPT_CHUNKS, CHUNK)
        dst_r1 = dst_flat.reshape(NW, EPT_CHUNKS, CHUNK)
    src_r = src_flat.reshape(NW, EPT_CHUNKS, CHUNK)
    dst_r = dst_flat.reshape(NW, EPT_CHUNKS, CHUNK)

    wcol = input_weight.reshape(N_NODES, 1)
    z = jnp.zeros((N_PAD, D_IN), jnp.float32)

    hw1 = _tc_scale(H, wcol, FSPLIT1)
    d1 = 64 if FSPLIT1 else 128
    p1 = _sc_scatter(hw1, z[:, :d1], src_r1, dst_r1, d1, FSPLIT1, MODE1)
    h1, hw2 = _tc_layer(p1, H, W1, b1.reshape(1, -1), wcol, FSPLIT1, "relu")
    p2 = _sc_scatter(hw2, z[:, :32], src_r, dst_r, 32, False, MODE23)
    h2, hw3 = _tc_layer(p2, h1, W2, b2.reshape(1, -1), wcol, False, "relu")
    p3 = _sc_scatter(hw3, z[:, :16], src_r, dst_r, 16, False, MODE23)
    return _tc_last(p3, h2, W3, b3.reshape(1, -1))
